# baseline (speedup 1.0000x reference)
import functools

import jax
import jax.numpy as jnp
from jax import lax
from jax.experimental import pallas as pl
from jax.experimental.pallas import tpu as pltpu
from jax.experimental.pallas import tpu_sc as plsc

VOCAB = 100000
D = 1024
B = 128
L = 32
N = B * L
NC, NS, LANES = 2, 16, 16
NW = NC * NS
ROWS_PER_W = N // NW
IDROWS = ROWS_PER_W // L
CHUNK = 16
NCHUNK = ROWS_PER_W // CHUNK
VPR = D // LANES
NBUF = 6


def _emb_body(ids_hbm, pidx_hbm, emb_hbm, pos_hbm, out_hbm,
              idx_v, pidx_v, pos_v,
              b0, b1, b2, b3, b4, b5,
              g0, g1, g2, g3, g4, g5,
              s0, s1, s2, s3, s4, s5):
    bufs = (b0, b1, b2, b3, b4, b5)
    gsems = (g0, g1, g2, g3, g4, g5)
    ssems = (s0, s1, s2, s3, s4, s5)
    wid = lax.axis_index("s") * NC + lax.axis_index("c")
    base = wid * ROWS_PER_W

    pltpu.sync_copy(ids_hbm.at[pl.ds(wid * IDROWS, IDROWS)], idx_v)
    pltpu.sync_copy(pidx_hbm, pidx_v)
    pos_cp = pltpu.async_copy(pos_hbm.at[pidx_v], pos_v, gsems[NBUF - 1])

    def gather(c):
        idx = idx_v.at[c // 2, pl.ds((c % 2) * CHUNK, CHUNK)]
        return pltpu.async_copy(emb_hbm.at[idx], bufs[c % NBUF], gsems[c % NBUF])

    gcp = [None] * NBUF
    scp = [None] * NBUF
    for c in range(NBUF - 1):
        gcp[c] = gather(c)
    pos_cp.wait()
    for c in range(NCHUNK):
        bi = c % NBUF
        buf = bufs[bi]
        gcp[bi].wait()

        def col(j, carry):
            sl = pl.ds(j * LANES, LANES)
            pv = pos_v[0, sl]
            for r in range(CHUNK):
                buf[r, sl] = buf[r, sl] + pv
            return carry

        lax.fori_loop(0, VPR, col, 0)
        scp[bi] = pltpu.async_copy(
            buf, out_hbm.at[pl.ds(base + c * CHUNK, CHUNK)], ssems[bi])

        nxt = c + NBUF - 1
        if nxt < NCHUNK:
            nb = nxt % NBUF
            if scp[nb] is not None:
                scp[nb].wait()
                scp[nb] = None
            gcp[nb] = gather(nxt)
    for cp in scp:
        if cp is not None:
            cp.wait()


_emb_kernel = functools.partial(
    pl.kernel,
    out_type=jax.ShapeDtypeStruct((N, D), jnp.float32),
    mesh=plsc.VectorSubcoreMesh(core_axis_name="c", subcore_axis_name="s",
                                num_cores=NC, num_subcores=NS),
    scratch_types=(
        [pltpu.VMEM((IDROWS, L), jnp.int32),
         pltpu.VMEM((1,), jnp.int32),
         pltpu.VMEM((1, D), jnp.float32)]
        + [pltpu.VMEM((CHUNK, D), jnp.float32)] * NBUF
        + [pltpu.SemaphoreType.DMA] * (2 * NBUF)
    ),
)(_emb_body)


def kernel(gpt_ids, gen_len, emb_table, pos_table):
    pidx = jnp.reshape(jnp.asarray(gen_len, jnp.int32), (1,))
    flat = _emb_kernel(gpt_ids.astype(jnp.int32), pidx, emb_table, pos_table)
    return jnp.reshape(flat, (B, L, D)), gen_len + 1
